# baseline (speedup 1.0000x reference)
import jax
import jax.numpy as jnp
from jax.experimental import pallas as pl
from jax.experimental.pallas import tpu as pltpu

_TSZ = 4096
_K = 8192
_D = 256
_GAMMA = 0.25

_BR = 256
_NI = _TSZ // _BR
_NJ = 2
_BCK = _K // _NJ


def _main_kernel(x_ref, e_ref, d_ref, a_ref, loss_ref,
                 zen_ref, en_ref, runm_ref, runa_ref, acc_ref):
    j = pl.program_id(0)
    i = pl.program_id(1)

    @pl.when(i == 0)
    def _():
        et = jnp.transpose(e_ref[:, 0, :])
        n1 = jnp.sqrt(jnp.sum(et * et, axis=0, keepdims=True))
        en_ref[...] = et / jnp.maximum(n1, 1e-12)

    @pl.when(jnp.logical_and(j == 0, i == 0))
    def _():
        ze = x_ref[0]
        an = jnp.sqrt(jnp.sum(ze * ze, axis=1, keepdims=True))
        zen_ref[...] = ze / jnp.maximum(an, 1e-8)

    rows = pl.ds(i * _BR, _BR)
    dt = jax.lax.dot_general(
        zen_ref[rows, :], en_ref[...], (((1,), (0,)), ((), ())),
        preferred_element_type=jnp.float32,
    )
    d_ref[...] = dt

    tile_max = jnp.max(dt, axis=1, keepdims=True)
    iota = jax.lax.broadcasted_iota(jnp.int32, dt.shape, 1)
    masked = jnp.where(dt == tile_max, iota, _K)
    tile_arg = jnp.min(masked, axis=1, keepdims=True) + j * _BCK

    @pl.when(j == 0)
    def _():
        runm_ref[rows, :] = tile_max
        runa_ref[rows, :] = tile_arg

    @pl.when(j > 0)
    def _():
        better = tile_max > runm_ref[rows, :]
        runa_ref[rows, :] = jnp.where(better, tile_arg, runa_ref[rows, :])
        runm_ref[rows, :] = jnp.maximum(tile_max, runm_ref[rows, :])

    @pl.when(j == _NJ - 1)
    def _():
        m = runm_ref[rows, :]
        a_ref[...] = runa_ref[rows, :]
        logp = jnp.maximum(jnp.log(jnp.clip(m, 1e-12, 1.0)), -100.0)
        part = jnp.sum(logp) * (-(1.0 + _GAMMA) / _TSZ)

        @pl.when(i == 0)
        def _():
            acc_ref[0] = part

        @pl.when(i > 0)
        def _():
            acc_ref[0] = acc_ref[0] + part

        @pl.when(i == _NI - 1)
        def _():
            loss_ref[...] = jnp.reshape(acc_ref[0], (1, 1))


def kernel(x, embedding):
    d, a, loss = pl.pallas_call(
        _main_kernel,
        grid=(_NJ, _NI),
        in_specs=[
            pl.BlockSpec((1, _TSZ, _D), lambda j, i: (0, 0, 0)),
            pl.BlockSpec((_BCK, 1, _D), lambda j, i: (j, 0, 0)),
        ],
        out_specs=[
            pl.BlockSpec((_BR, _BCK), lambda j, i: (i, j)),
            pl.BlockSpec((_BR, 1), lambda j, i: (i, 0)),
            pl.BlockSpec((1, 1), lambda j, i: (0, 0)),
        ],
        out_shape=[
            jax.ShapeDtypeStruct((_TSZ, _K), jnp.float32),
            jax.ShapeDtypeStruct((_TSZ, 1), jnp.int32),
            jax.ShapeDtypeStruct((1, 1), jnp.float32),
        ],
        scratch_shapes=[
            pltpu.VMEM((_TSZ, _D), jnp.float32),
            pltpu.VMEM((_D, _BCK), jnp.float32),
            pltpu.VMEM((_TSZ, 1), jnp.float32),
            pltpu.VMEM((_TSZ, 1), jnp.int32),
            pltpu.SMEM((1,), jnp.float32),
        ],
        compiler_params=pltpu.CompilerParams(
            dimension_semantics=("arbitrary", "arbitrary"),
        ),
    )(x, embedding)
    return d, a.reshape(_TSZ), loss.reshape(())


# speedup vs baseline: 1.1877x; 1.1877x over previous
import jax
import jax.numpy as jnp
from jax.experimental import pallas as pl
from jax.experimental.pallas import tpu as pltpu

_TSZ = 4096
_K = 8192
_D = 256
_GAMMA = 0.25

_BR = 256
_NI = _TSZ // _BR


def _main_kernel(x_ref, e_ref, d_ref, a_ref, loss_ref, en_ref, acc_ref):
    i = pl.program_id(0)

    @pl.when(i == 0)
    def _():
        et = jnp.transpose(e_ref[:, 0, :])
        n1 = jnp.sqrt(jnp.sum(et * et, axis=0, keepdims=True))
        en_ref[...] = et / jnp.maximum(n1, 1e-12)

    ze = x_ref[0, pl.ds(i * _BR, _BR), :]
    an = jnp.sqrt(jnp.sum(ze * ze, axis=1, keepdims=True))
    zen = ze / jnp.maximum(an, 1e-8)

    dt = jax.lax.dot_general(
        zen, en_ref[...], (((1,), (0,)), ((), ())),
        preferred_element_type=jnp.float32,
    )
    d_ref[...] = dt

    m = jnp.max(dt, axis=1, keepdims=True)
    iota = jax.lax.broadcasted_iota(jnp.int32, dt.shape, 1)
    masked = jnp.where(dt == m, iota, _K)
    a_ref[...] = jnp.min(masked, axis=1, keepdims=True)

    logp = jnp.maximum(jnp.log(jnp.clip(m, 1e-12, 1.0)), -100.0)
    part = jnp.sum(logp) * (-(1.0 + _GAMMA) / _TSZ)

    @pl.when(i == 0)
    def _():
        acc_ref[0] = part

    @pl.when(i > 0)
    def _():
        acc_ref[0] = acc_ref[0] + part

    @pl.when(i == _NI - 1)
    def _():
        loss_ref[...] = jnp.reshape(acc_ref[0], (1, 1))


def kernel(x, embedding):
    d, a, loss = pl.pallas_call(
        _main_kernel,
        grid=(_NI,),
        in_specs=[
            pl.BlockSpec((1, _TSZ, _D), lambda i: (0, 0, 0)),
            pl.BlockSpec((_K, 1, _D), lambda i: (0, 0, 0)),
        ],
        out_specs=[
            pl.BlockSpec((_BR, _K), lambda i: (i, 0)),
            pl.BlockSpec((_BR, 1), lambda i: (i, 0)),
            pl.BlockSpec((1, 1), lambda i: (0, 0)),
        ],
        out_shape=[
            jax.ShapeDtypeStruct((_TSZ, _K), jnp.float32),
            jax.ShapeDtypeStruct((_TSZ, 1), jnp.int32),
            jax.ShapeDtypeStruct((1, 1), jnp.float32),
        ],
        scratch_shapes=[
            pltpu.VMEM((_D, _K), jnp.float32),
            pltpu.SMEM((1,), jnp.float32),
        ],
        compiler_params=pltpu.CompilerParams(
            dimension_semantics=("arbitrary",),
        ),
    )(x, embedding)
    return d, a.reshape(_TSZ), loss.reshape(())
